# baseline (speedup 1.0000x reference)
import functools

import jax
import jax.numpy as jnp
from jax import lax
from jax.experimental import pallas as pl
from jax.experimental.pallas import tpu as pltpu

B, L, DIM_IN, DIM_OUT, H = 16, 4096, 64, 64, 256

_DN_T = (((1,), (1,)), ((), ()))


def _mm_t(a, w):
    return lax.dot_general(a, w, _DN_T, preferred_element_type=jnp.float32)


def _phi_kernel(x_ref, m_ref, w0_ref, b0_ref, w1_ref, b1_ref,
                acc_ref, cnt_ref):
    b = pl.program_id(0)

    m = m_ref[0].astype(jnp.bfloat16)

    b0r = b0_ref[...].reshape(1, H).astype(jnp.bfloat16)
    b1r = b1_ref[...].reshape(1, H).astype(jnp.bfloat16)

    xb = x_ref[0].astype(jnp.bfloat16).T
    h = _mm_t(xb, w0_ref[...].astype(jnp.bfloat16)).astype(jnp.bfloat16)
    h = jnp.maximum(h + b0r, jnp.bfloat16(0.0))
    h = _mm_t(h, w1_ref[...].astype(jnp.bfloat16)).astype(jnp.bfloat16)
    h = jnp.maximum(h + b1r, jnp.bfloat16(0.0))
    u = jnp.dot(m, h, preferred_element_type=jnp.float32)
    c = jnp.sum(m_ref[0].astype(jnp.float32))

    acc_ref[pl.ds(b, 1), :] = u
    cnt_ref[pl.ds(b, 1), :] = jnp.full((1, 128), c, jnp.float32)


def _rho_kernel(acc_ref, cnt_ref, w2_ref, b2_ref,
                rw0_ref, rb0_ref, rw1_ref, rb1_ref, rw2_ref, rb2_ref,
                out_ref):
    cnt = cnt_ref[:, 0:1]
    s = _mm_t(acc_ref[...], w2_ref[...]) + cnt * b2_ref[...].reshape(1, H)
    r = jnp.maximum(_mm_t(s, rw0_ref[...]) + rb0_ref[...].reshape(1, H), 0.0)
    r = jnp.maximum(_mm_t(r, rw1_ref[...]) + rb1_ref[...].reshape(1, H), 0.0)
    r = _mm_t(r, rw2_ref[...]) + rb2_ref[...].reshape(1, DIM_OUT)
    out_ref[...] = jnp.where(cnt > 0.0, r, 0.0)


@functools.partial(jax.jit, static_argnames=("interpret",))
def _run(x, mask, w0, b0, w1, b1, w2, b2, rw0, rb0, rw1, rb1, rw2, rb2,
         interpret=False):
    xt = jnp.transpose(x, (0, 2, 1))
    mf = mask.reshape(B, 1, L)

    fullb = lambda shape: pl.BlockSpec(shape, lambda b: (0,) * len(shape))
    acc, cnt = pl.pallas_call(
        _phi_kernel,
        grid=(B,),
        in_specs=[
            pl.BlockSpec((1, DIM_IN, L), lambda b: (b, 0, 0)),
            pl.BlockSpec((1, 1, L), lambda b: (b, 0, 0)),
            fullb((H, DIM_IN)), fullb((H,)),
            fullb((H, H)), fullb((H,)),
        ],
        out_specs=[fullb((B, H)), fullb((B, 128))],
        out_shape=[jax.ShapeDtypeStruct((B, H), jnp.float32),
                   jax.ShapeDtypeStruct((B, 128), jnp.float32)],
        compiler_params=pltpu.CompilerParams(
            dimension_semantics=("arbitrary",)),
        interpret=interpret,
    )(xt, mf, w0, b0, w1, b1)

    full1 = lambda shape: pl.BlockSpec(shape, lambda: (0,) * len(shape))
    return pl.pallas_call(
        _rho_kernel,
        in_specs=[
            full1((B, H)), full1((B, 128)),
            full1((H, H)), full1((H,)),
            full1((H, H)), full1((H,)),
            full1((H, H)), full1((H,)),
            full1((DIM_OUT, H)), full1((DIM_OUT,)),
        ],
        out_specs=full1((B, DIM_OUT)),
        out_shape=jax.ShapeDtypeStruct((B, DIM_OUT), jnp.float32),
        interpret=interpret,
    )(acc, cnt, w2, b2, rw0, rb0, rw1, rb1, rw2, rb2)


def kernel(x, mask, phi_w0, phi_b0, phi_w1, phi_b1, phi_w2, phi_b2,
           rho_w0, rho_b0, rho_w1, rho_b1, rho_w2, rho_b2):
    return _run(x, mask, phi_w0, phi_b0, phi_w1, phi_b1, phi_w2, phi_b2,
                rho_w0, rho_b0, rho_w1, rho_b1, rho_w2, rho_b2)
